# baseline (speedup 1.0000x reference)
import functools

import jax
import jax.numpy as jnp
from jax import lax
from jax.experimental import pallas as pl
from jax.experimental.pallas import tpu as pltpu
from jax.experimental.pallas import tpu_sc as plsc

EMBED_DIM = 64
PAD_DIM = 128


def kernel(heads, relations, tails, entity_embeddings, relation_embeddings):
    B = heads.shape[0]
    E, D = entity_embeddings.shape
    assert D == EMBED_DIM

    info = plsc.get_sparse_core_info()
    NC, NS, L = info.num_cores, info.num_subcores, info.num_lanes
    NW = NC * NS
    assert B % (8 * NW) == 0
    bpw = B // NW
    CH = bpw // 2

    NFULL = E // PAD_DIM
    TAIL = E - NFULL * PAD_DIM
    per_w = NFULL // NW + (1 if NFULL % NW else 0)

    mesh = plsc.VectorSubcoreMesh(core_axis_name="c", subcore_axis_name="s")

    @functools.partial(
        pl.kernel,
        mesh=mesh,
        out_type=jax.ShapeDtypeStruct((E, PAD_DIM), jnp.float32),
        compiler_params=pltpu.CompilerParams(needs_layout_passes=False),
        scratch_types=[
            pltpu.VMEM((D, PAD_DIM), jnp.float32),
            pltpu.VMEM((PAD_DIM, PAD_DIM), jnp.float32),
        ],
    )
    def relayout(entT_hbm, tail_hbm, out_hbm, in_v, tr_v):
        wid = lax.axis_index("s") * NC + lax.axis_index("c")
        lane = lax.iota(jnp.int32, L)
        diag = [((s + lane) & (L - 1)) for s in range(L)]

        def do_block(j):
            col0 = pl.multiple_of(j * PAD_DIM, PAD_DIM)
            pltpu.sync_copy(entT_hbm.at[:, pl.ds(col0, PAD_DIM)], in_v)

            def egroup(g, carry):
                for q in range(D // L):
                    dvec = q * L + lane
                    for s in range(L):
                        evec = g * L + diag[s]
                        val = plsc.load_gather(in_v, [dvec, evec])
                        plsc.store_scatter(tr_v, [evec, dvec], val)
                return carry

            lax.fori_loop(0, PAD_DIM // L, egroup, 0)
            pltpu.sync_copy(tr_v, out_hbm.at[pl.ds(col0, PAD_DIM)])

        def body(i, carry):
            j = wid * per_w + i

            @pl.when(j < NFULL)
            def _():
                do_block(j)

            return carry

        lax.fori_loop(0, per_w, body, 0)

        if TAIL:
            @pl.when(wid == NW - 1)
            def _():
                pltpu.sync_copy(tail_hbm, tr_v.at[pl.ds(0, TAIL)])
                pltpu.sync_copy(tr_v.at[pl.ds(0, TAIL)],
                                out_hbm.at[pl.ds(NFULL * PAD_DIM, TAIL)])

    @functools.partial(
        pl.kernel,
        mesh=mesh,
        out_type=jax.ShapeDtypeStruct((B,), jnp.float32),
        compiler_params=pltpu.CompilerParams(needs_layout_passes=False),
        scratch_types=[
            pltpu.VMEM((bpw,), jnp.int32),
            pltpu.VMEM((bpw,), jnp.int32),
            pltpu.VMEM((bpw,), jnp.int32),
            pltpu.VMEM((CH, PAD_DIM), jnp.float32),
            pltpu.VMEM((CH, PAD_DIM), jnp.float32),
            pltpu.VMEM((CH, PAD_DIM), jnp.float32),
            pltpu.VMEM((bpw,), jnp.float32),
            pltpu.SemaphoreType.DMA,
        ],
    )
    def trans_e(heads_hbm, rel_hbm, tails_hbm, ent_hbm, relemb_hbm, out_hbm,
                hid_v, rid_v, tid_v, h_v, r_v, t_v, o_v, sem):
        wid = lax.axis_index("s") * NC + lax.axis_index("c")
        base = wid * bpw

        pltpu.sync_copy(heads_hbm.at[pl.ds(base, bpw)], hid_v)
        pltpu.sync_copy(rel_hbm.at[pl.ds(base, bpw)], rid_v)
        pltpu.sync_copy(tails_hbm.at[pl.ds(base, bpw)], tid_v)

        lane = lax.iota(jnp.int32, L)
        last_lane = lane == (L - 1)

        for c in range(bpw // CH):
            csl = pl.ds(c * CH, CH)
            ch = pltpu.async_copy(ent_hbm.at[hid_v.at[csl]], h_v, sem)
            cr = pltpu.async_copy(relemb_hbm.at[rid_v.at[csl]], r_v, sem)
            ct = pltpu.async_copy(ent_hbm.at[tid_v.at[csl]], t_v, sem)
            ch.wait()
            cr.wait()
            ct.wait()

            def row(i, carry):
                acc = jnp.zeros((L,), jnp.float32)
                for q in range(D // L):
                    sl = pl.ds(q * L, L)
                    dv = h_v[i, sl] + r_v[i, sl] - t_v[i, sl]
                    acc = acc + dv * dv
                s = plsc.cumsum(acc)
                plsc.store_scatter(
                    o_v, [jnp.full((L,), c * CH + i, jnp.int32)], s,
                    mask=last_lane)
                return carry

            lax.fori_loop(0, CH, row, 0)

        def newton_sqrt(i, carry):
            sl = pl.ds(i * L, L)
            x = o_v[sl]
            xi = plsc.bitcast(x, jnp.int32)
            y = plsc.bitcast((xi >> 1) + jnp.int32(0x1FBD1DF5), jnp.float32)
            y = 0.5 * (y + x / y)
            y = 0.5 * (y + x / y)
            y = 0.5 * (y + x / y)
            o_v[sl] = y
            return carry

        lax.fori_loop(0, bpw // L, newton_sqrt, 0)

        pltpu.sync_copy(o_v, out_hbm.at[pl.ds(base, bpw)])

    tail_pad = jnp.pad(entity_embeddings[NFULL * PAD_DIM:],
                       ((0, 0), (0, PAD_DIM - D)))
    ent_pad = relayout(entity_embeddings.T, tail_pad)
    rel_pad = jnp.pad(relation_embeddings, ((0, 0), (0, PAD_DIM - D)))
    return trans_e(heads, relations, tails, ent_pad, rel_pad)
